# Initial kernel scaffold; baseline (speedup 1.0000x reference)
import jax
import jax.numpy as jnp
from jax.experimental import pallas as pl


def kernel(data_in, times_in, mask_in, lengths, params):
    raise NotImplementedError("write your pallas kernel here")


# speedup vs baseline: 3.7539x; 3.7539x over previous
import functools

import jax
import jax.numpy as jnp
import numpy as np
from jax.experimental import pallas as pl
from jax.experimental.pallas import tpu as pltpu

V = 36
D_OB = 4
T = 215
B = 128
VD = V * D_OB
D = T * D_OB
DIM_POS = 16
E_ENC = VD + DIM_POS
NHEAD = 8
DH = E_ENC // NHEAD
NHID = 512

_TS = np.power(float(T), np.linspace(0.0, 1.0, DIM_POS // 2)).astype(np.float32)


def _dil_body(x_ref, w_ref, o_ref):
    o_ref[...] = jnp.maximum(
        jnp.dot(x_ref[...], w_ref[...], preferred_element_type=jnp.float32), 0.0)


def _mm_body(x_ref, w_ref, o_ref):
    o_ref[...] = jnp.dot(x_ref[...], w_ref[...], preferred_element_type=jnp.float32)


def _matmul(x, w, body, bm):
    m, k = x.shape
    _, n = w.shape
    grid = m // bm
    return pl.pallas_call(
        body,
        grid=(grid,),
        in_specs=[
            pl.BlockSpec((bm, k), lambda i: (i, 0)),
            pl.BlockSpec((k, n), lambda i: (0, 0)),
        ],
        out_specs=pl.BlockSpec((bm, n), lambda i: (i, 0)),
        out_shape=jax.ShapeDtypeStruct((m, n), jnp.float32),
    )(x, w)


def _gat1_body(qkv_ref, o_ref, a_ref):
    q = qkv_ref[0, 0]
    k = qkv_ref[0, 1]
    v = qkv_ref[0, 2]
    l = jax.lax.dot_general(k, q, (((1,), (1,)), ((), ())),
                            preferred_element_type=jnp.float32)
    l = l / jnp.sqrt(jnp.float32(D))
    m = jnp.max(l, axis=0, keepdims=True)
    e = jnp.exp(l - m)
    s = jnp.sum(e, axis=0, keepdims=True)
    a = e / (s + 1e-16)
    o_ref[0] = jax.lax.dot_general(a, v, (((0,), (0,)), ((), ())),
                                   preferred_element_type=jnp.float32)
    a_ref[0] = a


def _gat2_body(qkv_ref, ew_ref, o_ref, a_ref):
    q = qkv_ref[0, 0]
    k = qkv_ref[0, 1]
    v = qkv_ref[0, 2]
    l = jax.lax.dot_general(k, q, (((1,), (1,)), ((), ())),
                            preferred_element_type=jnp.float32)
    l = l / jnp.sqrt(jnp.float32(D)) * ew_ref[0]
    m = jnp.max(l, axis=0, keepdims=True)
    e = jnp.exp(l - m)
    s = jnp.sum(e, axis=0, keepdims=True)
    a = e / (s + 1e-16)
    o_ref[0] = jax.lax.dot_general(a, v, (((0,), (0,)), ((), ())),
                                   preferred_element_type=jnp.float32)
    a_ref[0] = a


def _gat_attn(qkv, ew):
    out_shape = (jax.ShapeDtypeStruct((B, V, D), jnp.float32),
                 jax.ShapeDtypeStruct((B, V, V), jnp.float32))
    qkv_spec = pl.BlockSpec((1, 3, V, D), lambda b: (b, 0, 0, 0))
    o_spec = pl.BlockSpec((1, V, D), lambda b: (b, 0, 0))
    a_spec = pl.BlockSpec((1, V, V), lambda b: (b, 0, 0))
    if ew is None:
        return pl.pallas_call(
            _gat1_body, grid=(B,),
            in_specs=[qkv_spec],
            out_specs=(o_spec, a_spec),
            out_shape=out_shape,
        )(qkv)
    return pl.pallas_call(
        _gat2_body, grid=(B,),
        in_specs=[qkv_spec, a_spec],
        out_specs=(o_spec, a_spec),
        out_shape=out_shape,
    )(qkv, ew)


def _pe_body(t_ref, ts_ref, o_ref):
    x = t_ref[...]
    ts = ts_ref[...]
    scaled = x[:, None, :] / ts[None, :, :]
    o_ref[...] = jnp.concatenate([jnp.sin(scaled), jnp.cos(scaled)], axis=1)


def _pos_encode(times_b):
    return pl.pallas_call(
        _pe_body, grid=(B // 8,),
        in_specs=[pl.BlockSpec((8, T), lambda i: (i, 0)),
                  pl.BlockSpec((DIM_POS // 2, 1), lambda i: (0, 0))],
        out_specs=pl.BlockSpec((8, DIM_POS, T), lambda i: (i, 0, 0)),
        out_shape=jax.ShapeDtypeStruct((B, DIM_POS, T), jnp.float32),
    )(times_b, jnp.asarray(_TS).reshape(DIM_POS // 2, 1))


def _dist_body(a_ref, o_ref):
    a = a_ref[...]

    def body(i, acc):
        row = a_ref[pl.ds(i, 1), :]
        d = a - row
        s = jnp.sum(d * d, axis=1, keepdims=True) + 1e-12
        return acc + jnp.sum(jnp.sqrt(s))

    acc = jax.lax.fori_loop(0, B, body, jnp.float32(0.0))
    o_ref[0, 0] = acc / jnp.float32(B * B)


def _distance(a2):
    out = pl.pallas_call(
        _dist_body,
        in_specs=[pl.BlockSpec((B, V * V), lambda: (0, 0))],
        out_specs=pl.BlockSpec(memory_space=pltpu.SMEM),
        out_shape=jax.ShapeDtypeStruct((1, 1), jnp.float32),
    )(a2)
    return out[0, 0]


def _ln(x):
    m = jnp.mean(x, axis=-1, keepdims=True)
    v = jnp.mean((x - m) * (x - m), axis=-1, keepdims=True)
    return (x - m) / jnp.sqrt(v + 1e-5)


def _enc_body(x_ref, pad_ref, wq_ref, wk_ref, wv_ref, wo_ref, w1_ref, w2_ref,
              o_ref):
    x = x_ref[0]
    pad = pad_ref[0] > 0.5
    attn = jnp.zeros((T, E_ENC), jnp.float32)
    for h in range(NHEAD):
        qh = jnp.dot(x, wq_ref[h], preferred_element_type=jnp.float32)
        kh = jnp.dot(x, wk_ref[h], preferred_element_type=jnp.float32)
        vh = jnp.dot(x, wv_ref[h], preferred_element_type=jnp.float32)
        lg = jax.lax.dot_general(qh, kh, (((1,), (1,)), ((), ())),
                                 preferred_element_type=jnp.float32)
        lg = lg / jnp.sqrt(jnp.float32(DH))
        lg = jnp.where(pad, jnp.float32(-1e9), lg)
        mx = jnp.max(lg, axis=1, keepdims=True)
        ex = jnp.exp(lg - mx)
        al = ex / jnp.sum(ex, axis=1, keepdims=True)
        oh = jnp.dot(al, vh, preferred_element_type=jnp.float32)
        attn = attn + jnp.dot(oh, wo_ref[h], preferred_element_type=jnp.float32)
    x1 = _ln(x + attn)
    ff = jnp.dot(
        jnp.maximum(jnp.dot(x1, w1_ref[...], preferred_element_type=jnp.float32),
                    0.0),
        w2_ref[...], preferred_element_type=jnp.float32)
    o_ref[0] = _ln(x1 + ff)


def _enc_layer(x, padf, p):
    wq = p["Wq"].reshape(E_ENC, NHEAD, DH).transpose(1, 0, 2)
    wk = p["Wk"].reshape(E_ENC, NHEAD, DH).transpose(1, 0, 2)
    wv = p["Wv"].reshape(E_ENC, NHEAD, DH).transpose(1, 0, 2)
    wo = p["Wo"].reshape(NHEAD, DH, E_ENC)
    full = lambda shape: pl.BlockSpec(shape, lambda b: (0,) * len(shape))
    return pl.pallas_call(
        _enc_body, grid=(B,),
        in_specs=[
            pl.BlockSpec((1, T, E_ENC), lambda b: (b, 0, 0)),
            pl.BlockSpec((1, 1, T), lambda b: (b, 0, 0)),
            full((NHEAD, E_ENC, DH)),
            full((NHEAD, E_ENC, DH)),
            full((NHEAD, E_ENC, DH)),
            full((NHEAD, DH, E_ENC)),
            full((E_ENC, NHID)),
            full((NHID, E_ENC)),
        ],
        out_specs=pl.BlockSpec((1, T, E_ENC), lambda b: (b, 0, 0)),
        out_shape=jax.ShapeDtypeStruct((B, T, E_ENC), jnp.float32),
    )(x, padf, wq, wk, wv, wo, p["W1"], p["W2"])


def kernel(data_in, times_in, mask_in, lengths, params):
    wc = params["W_dil"].reshape(V, D_OB, VD).sum(axis=1)
    h = _matmul(data_in.reshape(T * B, V), wc, _dil_body, bm=T * B // 8)
    x = h.reshape(T, B, V, D_OB).transpose(1, 2, 0, 3).reshape(B * V, D)

    wqkv1 = jnp.concatenate([params["Wq1"], params["Wk1"], params["Wv1"]], axis=1)
    wqkv2 = jnp.concatenate([params["Wq2"], params["Wk2"], params["Wv2"]], axis=1)

    qkv1 = _matmul(x, wqkv1, _mm_body, bm=512)
    qkv1 = qkv1.reshape(B, V, 3, D).transpose(0, 2, 1, 3)
    out1, _a1 = _gat_attn(qkv1, None)

    qkv2 = _matmul(out1.reshape(B * V, D), wqkv2, _mm_body, bm=512)
    qkv2 = qkv2.reshape(B, V, 3, D).transpose(0, 2, 1, 3)
    out2, a2 = _gat_attn(qkv2, _a1)

    distance = _distance(a2.reshape(B, V * V))

    out_b = out2.reshape(B, V, T, D_OB).transpose(0, 2, 1, 3).reshape(B, T, VD)
    pe = _pos_encode(times_in.T).transpose(0, 2, 1)
    xenc = jnp.concatenate([out_b, pe], axis=2)
    padf = (jnp.arange(T)[None, :] >= lengths[:, None]).astype(
        jnp.float32).reshape(B, 1, T)

    for p in params["layers"]:
        xenc = _enc_layer(xenc, padf, p)

    return xenc.transpose(1, 0, 2), distance
